# baseline (speedup 1.0000x reference)
import jax
import jax.numpy as jnp
from jax.experimental import pallas as pl

_N = 1024 * 1024
_ROWS = 1024
_COLS = 1024
_BLOCK_ROWS = 128


def _gen_kernel(w_ref, n_ref, o_ref):
    o_ref[...] = jax.nn.sigmoid((w_ref[...] - n_ref[...]) * 10.0)


def kernel(weights, noises):
    w = weights.reshape(_ROWS, _COLS)
    n = noises.reshape(_ROWS, _COLS)
    out = pl.pallas_call(
        _gen_kernel,
        out_shape=jax.ShapeDtypeStruct((_ROWS, _COLS), jnp.float32),
        grid=(_ROWS // _BLOCK_ROWS,),
        in_specs=[
            pl.BlockSpec((_BLOCK_ROWS, _COLS), lambda i: (i, 0)),
            pl.BlockSpec((_BLOCK_ROWS, _COLS), lambda i: (i, 0)),
        ],
        out_specs=pl.BlockSpec((_BLOCK_ROWS, _COLS), lambda i: (i, 0)),
    )(w, n)
    return out.reshape(_N)


# speedup vs baseline: 1.1057x; 1.1057x over previous
import jax
import jax.numpy as jnp
from jax.experimental import pallas as pl

_N = 1024 * 1024
_ROWS = 1024
_COLS = 1024
_BLOCK_ROWS = 256


def _gen_kernel(w_ref, n_ref, o_ref):
    o_ref[...] = jax.nn.sigmoid((w_ref[...] - n_ref[...]) * 10.0)


def kernel(weights, noises):
    w = weights.reshape(_ROWS, _COLS)
    n = noises.reshape(_ROWS, _COLS)
    out = pl.pallas_call(
        _gen_kernel,
        out_shape=jax.ShapeDtypeStruct((_ROWS, _COLS), jnp.float32),
        grid=(_ROWS // _BLOCK_ROWS,),
        in_specs=[
            pl.BlockSpec((_BLOCK_ROWS, _COLS), lambda i: (i, 0)),
            pl.BlockSpec((_BLOCK_ROWS, _COLS), lambda i: (i, 0)),
        ],
        out_specs=pl.BlockSpec((_BLOCK_ROWS, _COLS), lambda i: (i, 0)),
    )(w, n)
    return out.reshape(_N)


# speedup vs baseline: 3.4047x; 3.0792x over previous
import jax
import jax.numpy as jnp
from jax.experimental import pallas as pl

_N = 1024 * 1024
_STEPS = 4
_BLOCK = _N // _STEPS


def _gen_kernel(w_ref, n_ref, o_ref):
    o_ref[...] = jax.nn.sigmoid((w_ref[...] - n_ref[...]) * 10.0)


def kernel(weights, noises):
    return pl.pallas_call(
        _gen_kernel,
        out_shape=jax.ShapeDtypeStruct((_N,), jnp.float32),
        grid=(_STEPS,),
        in_specs=[
            pl.BlockSpec((_BLOCK,), lambda i: (i,)),
            pl.BlockSpec((_BLOCK,), lambda i: (i,)),
        ],
        out_specs=pl.BlockSpec((_BLOCK,), lambda i: (i,)),
    )(weights, noises)
